# baseline (speedup 1.0000x reference)
import functools

import jax
import jax.numpy as jnp
from jax import lax
from jax.experimental import pallas as pl
from jax.experimental.pallas import tpu as pltpu
from jax.experimental.pallas import tpu_sc as plsc

_B = 16384
_D = 32
_H = 256
_O = 128


_NC = 2
_NS = 16
_NW = _NC * _NS
_B_PER_W = _B // _NW


@functools.cache
def _make_sc_gather():
    @functools.partial(
        pl.kernel,
        out_type=jax.ShapeDtypeStruct((_B, _D), jnp.float32),
        mesh=plsc.VectorSubcoreMesh(core_axis_name="c", subcore_axis_name="s"),
        scratch_types=[
            pltpu.VMEM((_B_PER_W,), jnp.int32),
            pltpu.VMEM((_B_PER_W, _D), jnp.float32),
            pltpu.SemaphoreType.DMA,
        ],
        compiler_params=pltpu.CompilerParams(use_tc_tiling_on_sc=False),
    )
    def _sc_gather(table_hbm, idx_hbm, out_hbm, idx_v, rows_v, sem):
        wid = lax.axis_index("s") * _NC + lax.axis_index("c")
        base = wid * _B_PER_W
        pltpu.sync_copy(idx_hbm.at[pl.ds(base, _B_PER_W)], idx_v)
        pltpu.async_copy(table_hbm.at[idx_v], rows_v, sem).wait()
        pltpu.sync_copy(rows_v, out_hbm.at[pl.ds(base, _B_PER_W)])

    return _sc_gather


_BM = 2048


def _mlp_body(h_ref, w1_ref, b1_ref, w2_ref, b2_ref, wo_ref, bo_ref, out_ref):
    h = h_ref[...]
    a = jnp.dot(h, w1_ref[...], preferred_element_type=jnp.float32)
    a = jnp.maximum(a + b1_ref[...], 0.0)
    a = jnp.dot(a, w2_ref[...], preferred_element_type=jnp.float32)
    a = jnp.maximum(a + b2_ref[...], 0.0)
    a = jnp.dot(a, wo_ref[...], preferred_element_type=jnp.float32)
    out_ref[...] = a + bo_ref[...]


def _mlp(h, W1, b1, W2, b2, Wout, bout):
    grid = (_B // _BM,)
    full = lambda i: (0, 0)
    return pl.pallas_call(
        _mlp_body,
        grid=grid,
        in_specs=[
            pl.BlockSpec((_BM, _D), lambda i: (i, 0)),
            pl.BlockSpec((_D, _H), full),
            pl.BlockSpec((1, _H), full),
            pl.BlockSpec((_H, _H), full),
            pl.BlockSpec((1, _H), full),
            pl.BlockSpec((_H, _O), full),
            pl.BlockSpec((1, _O), full),
        ],
        out_specs=pl.BlockSpec((_BM, _O), lambda i: (i, 0)),
        out_shape=jax.ShapeDtypeStruct((_B, _O), jnp.float32),
        compiler_params=pltpu.CompilerParams(
            dimension_semantics=("parallel",),
        ),
    )(h, W1, b1, W2, b2, Wout, bout)


def kernel(x, emb, W1, b1, W2, b2, Wout, bout):
    h = _make_sc_gather()(emb, x.astype(jnp.int32))
    return _mlp(
        h,
        W1,
        b1.reshape(1, _H),
        W2,
        b2.reshape(1, _H),
        Wout,
        bout.reshape(1, _O),
    )
